# baseline (speedup 1.0000x reference)
import functools

import jax
import jax.numpy as jnp
from jax import lax
from jax.experimental import pallas as pl
from jax.experimental.pallas import tpu as pltpu
from jax.experimental.pallas import tpu_sc as plsc

N = 10000
E = 320000
D = 128
H = 64
C = 32
G = 64

NC = 2
NS = 16
NW = NC * NS
NPAD = 10240
RPS = NPAD // NS
CHB = 128
NCB = 80
IB = 16
EW = NCB * CHB
EPAD = NW * EW

R = 2048
NB = NPAD // R

_mesh = plsc.VectorSubcoreMesh(core_axis_name="c", subcore_axis_name="s")


@functools.partial(
    pl.kernel,
    out_type=jax.ShapeDtypeStruct((NC, NPAD), jnp.float32),
    mesh=_mesh,
    scratch_types=[
        pltpu.VMEM((NCB, CHB), jnp.int32),
        pltpu.VMEM((CHB,), jnp.float32),
        pltpu.VMEM((RPS,), jnp.float32),
        pltpu.VMEM_SHARED((NPAD,), jnp.float32),
    ],
)
def _deg_sc(dst_hbm, out_hbm, dstb_v, ones_v, zbuf_v, deg_sh):
    c = lax.axis_index("c")
    s = lax.axis_index("s")
    wid = c * NS + s

    pltpu.sync_copy(dst_hbm.at[pl.ds(wid * NCB, NCB), :], dstb_v)

    def _fill_ones(i, _):
        ones_v[pl.ds(i * 16, 16)] = jnp.ones((16,), jnp.float32)
        return 0

    lax.fori_loop(0, CHB // 16, _fill_ones, 0)

    def _fill_zero(i, _):
        zbuf_v[pl.ds(i * 16, 16)] = jnp.zeros((16,), jnp.float32)
        return 0

    lax.fori_loop(0, RPS // 16, _fill_zero, 0)
    pltpu.sync_copy(zbuf_v, deg_sh.at[pl.ds(s * RPS, RPS)])
    plsc.subcore_barrier()

    def _step(g, _):
        pltpu.sync_copy(ones_v, deg_sh.at[dstb_v.at[g]], add=True)
        return 0

    lax.fori_loop(0, NCB, _step, 0)
    plsc.subcore_barrier()
    pltpu.sync_copy(deg_sh.at[pl.ds(s * RPS, RPS)],
                    out_hbm.at[c, pl.ds(s * RPS, RPS)])


def _make_scatter(dk):
    @functools.partial(
        pl.kernel,
        out_type=jax.ShapeDtypeStruct((NC, NPAD, dk), jnp.float32),
        mesh=_mesh,
        compiler_params=pltpu.CompilerParams(
            use_tc_tiling_on_sc=(dk % 128 == 0)),
        scratch_types=[
            pltpu.VMEM((IB, CHB), jnp.int32),
            pltpu.VMEM((IB, CHB), jnp.int32),
            pltpu.VMEM((CHB, dk), jnp.float32),
            pltpu.VMEM((CHB, dk), jnp.float32),
            pltpu.VMEM_SHARED((NPAD, dk), jnp.float32),
            pltpu.SemaphoreType.DMA,
            pltpu.SemaphoreType.DMA,
        ],
    )
    def _scat(y_hbm, src_hbm, dst_hbm, out_hbm,
              srcb_v, dstb_v, buf0, buf1, acc_sh, sem0, sem1):
        c = lax.axis_index("c")
        s = lax.axis_index("s")
        wid = c * NS + s

        def _zrow(i, _):
            for j in range(dk // 16):
                buf0[i, pl.ds(j * 16, 16)] = jnp.zeros((16,), jnp.float32)
            return 0

        lax.fori_loop(0, CHB, _zrow, 0)
        for k in range(RPS // CHB):
            pltpu.sync_copy(buf0, acc_sh.at[pl.ds(s * RPS + k * CHB, CHB)])
        plsc.subcore_barrier()

        def _blk(b, _):
            pltpu.sync_copy(src_hbm.at[pl.ds(wid * NCB + b * IB, IB), :],
                            srcb_v)
            pltpu.sync_copy(dst_hbm.at[pl.ds(wid * NCB + b * IB, IB), :],
                            dstb_v)
            def _step(t, _):
                pltpu.async_copy(y_hbm.at[srcb_v.at[t]], buf0, sem0).wait()
                pltpu.sync_copy(buf0, acc_sh.at[dstb_v.at[t]], add=True)
                return 0

            lax.fori_loop(0, IB, _step, 0)
            return 0

        lax.fori_loop(0, NCB // IB, _blk, 0)
        plsc.subcore_barrier()
        pltpu.sync_copy(acc_sh.at[pl.ds(s * RPS, RPS)],
                        out_hbm.at[c, pl.ds(s * RPS, RPS)])

    return _scat


_scatter128 = _make_scatter(D)
_scatter64 = _make_scatter(H)


def _ka_body(deg_ref, x_ref, w1_ref, y1_ref, dinv_ref):
    deg = deg_ref[...]
    dinv = lax.rsqrt(deg[:, 0:1] + deg[:, 1:2] + 1.0)
    xw = jnp.dot(x_ref[...], w1_ref[...], preferred_element_type=jnp.float32)
    y1_ref[...] = dinv * xw
    dinv_ref[...] = dinv


def _ka(degt, x_pad, W1):
    return pl.pallas_call(
        _ka_body,
        grid=(NB,),
        in_specs=[
            pl.BlockSpec((R, 2), lambda i: (i, 0)),
            pl.BlockSpec((R, D), lambda i: (i, 0)),
            pl.BlockSpec((D, D), lambda i: (0, 0)),
        ],
        out_specs=[
            pl.BlockSpec((R, D), lambda i: (i, 0)),
            pl.BlockSpec((R, 1), lambda i: (i, 0)),
        ],
        out_shape=[
            jax.ShapeDtypeStruct((NPAD, D), jnp.float32),
            jax.ShapeDtypeStruct((NPAD, 1), jnp.float32),
        ],
    )(degt, x_pad, W1)


def _kb_body(p_ref, y1_ref, dinv_ref, b1_ref, w2_ref, y2_ref):
    p = p_ref[...]
    dinv = dinv_ref[...]
    h1 = jnp.maximum(dinv * (p[0] + p[1] + y1_ref[...]) + b1_ref[...], 0.0)
    y2_ref[...] = dinv * jnp.dot(h1, w2_ref[...],
                                 preferred_element_type=jnp.float32)


def _kb(p, y1, dinv, b1r, W2):
    return pl.pallas_call(
        _kb_body,
        grid=(NB,),
        in_specs=[
            pl.BlockSpec((NC, R, D), lambda i: (0, i, 0)),
            pl.BlockSpec((R, D), lambda i: (i, 0)),
            pl.BlockSpec((R, 1), lambda i: (i, 0)),
            pl.BlockSpec((1, D), lambda i: (0, 0)),
            pl.BlockSpec((D, H), lambda i: (0, 0)),
        ],
        out_specs=pl.BlockSpec((R, H), lambda i: (i, 0)),
        out_shape=jax.ShapeDtypeStruct((NPAD, H), jnp.float32),
    )(p, y1, dinv, b1r, W2)


def _kc_body(q_ref, y2_ref, dinv_ref, b2_ref, batch_ref, w3_ref, b3_ref,
             out_ref, psum):
    i = pl.program_id(0)

    @pl.when(i == 0)
    def _():
        psum[...] = jnp.zeros_like(psum)

    q = q_ref[...]
    dinv = dinv_ref[...]
    h2 = jnp.maximum(dinv * (q[0] + q[1] + y2_ref[...]) + b2_ref[...], 0.0)
    bb = batch_ref[...]
    gid = lax.broadcasted_iota(jnp.int32, (1, G), 1)
    m = (bb == gid).astype(jnp.float32)
    haug = jnp.concatenate([h2, jnp.ones((R, 1), jnp.float32)], axis=1)
    psum[...] += lax.dot_general(m, haug, (((0,), (0,)), ((), ())),
                                 preferred_element_type=jnp.float32)

    @pl.when(i == pl.num_programs(0) - 1)
    def _():
        ps = psum[...]
        pooled = ps[:, :H] / jnp.maximum(ps[:, H:H + 1], 1.0)
        out_ref[...] = jnp.dot(pooled, w3_ref[...],
                               preferred_element_type=jnp.float32) + b3_ref[...]


def _kc(q, y2, dinv, b2r, batch_pad, W3, b3r):
    return pl.pallas_call(
        _kc_body,
        grid=(NB,),
        in_specs=[
            pl.BlockSpec((NC, R, H), lambda i: (0, i, 0)),
            pl.BlockSpec((R, H), lambda i: (i, 0)),
            pl.BlockSpec((R, 1), lambda i: (i, 0)),
            pl.BlockSpec((1, H), lambda i: (0, 0)),
            pl.BlockSpec((R, 1), lambda i: (i, 0)),
            pl.BlockSpec((H, C), lambda i: (0, 0)),
            pl.BlockSpec((1, C), lambda i: (0, 0)),
        ],
        out_specs=pl.BlockSpec((G, C), lambda i: (0, 0)),
        out_shape=jax.ShapeDtypeStruct((G, C), jnp.float32),
        scratch_shapes=[pltpu.VMEM((G, H + 1), jnp.float32)],
    )(q, y2, dinv, b2r, batch_pad, W3, b3r)


def kernel(x, edge_index, batch, W1, b1, W2, b2, W3, b3):
    src = edge_index[0]
    dst = edge_index[1]
    srcp = jnp.pad(src, (0, EPAD - E),
                   constant_values=NPAD - 1).reshape(NW * NCB, CHB)
    dstp = jnp.pad(dst, (0, EPAD - E),
                   constant_values=NPAD - 1).reshape(NW * NCB, CHB)
    x_pad = jnp.pad(x, ((0, NPAD - N), (0, 0)))
    batch_pad = jnp.pad(batch, (0, NPAD - N), constant_values=G)
    batch_pad = batch_pad.reshape(NPAD, 1)

    degp = _deg_sc(dstp)
    degt = degp.T
    y1, dinv = _ka(degt, x_pad, W1)
    p = _scatter128(y1, srcp, dstp)
    y2 = _kb(p, y1, dinv, b1.reshape(1, D), W2)
    q = _scatter64(y2, srcp, dstp)
    return _kc(q, y2, dinv, b2.reshape(1, H), batch_pad, W3,
               b3.reshape(1, C))


# speedup vs baseline: 2.9214x; 2.9214x over previous
import functools

import jax
import jax.numpy as jnp
from jax import lax
from jax.experimental import pallas as pl
from jax.experimental.pallas import tpu as pltpu
from jax.experimental.pallas import tpu_sc as plsc

N = 10000
E = 320000
D = 128
H = 64
C = 32
G = 64

NC = 2
NS = 16
NW = NC * NS
NPAD = 10240
RPS = NPAD // NS
CHB = 128
NCB = 80
IB = 16
EW = NCB * CHB
EPAD = NW * EW

R = 2048
NB = NPAD // R

_mesh = plsc.VectorSubcoreMesh(core_axis_name="c", subcore_axis_name="s")


@functools.partial(
    pl.kernel,
    out_type=jax.ShapeDtypeStruct((NC, NPAD), jnp.float32),
    mesh=_mesh,
    scratch_types=[
        pltpu.VMEM((NCB, CHB), jnp.int32),
        pltpu.VMEM((CHB,), jnp.float32),
        pltpu.VMEM((RPS,), jnp.float32),
        pltpu.VMEM_SHARED((NPAD,), jnp.float32),
    ],
)
def _deg_sc(dst_hbm, out_hbm, dstb_v, ones_v, zbuf_v, deg_sh):
    c = lax.axis_index("c")
    s = lax.axis_index("s")
    wid = c * NS + s

    pltpu.sync_copy(dst_hbm.at[pl.ds(wid * NCB, NCB), :], dstb_v)

    def _fill_ones(i, _):
        ones_v[pl.ds(i * 16, 16)] = jnp.ones((16,), jnp.float32)
        return 0

    lax.fori_loop(0, CHB // 16, _fill_ones, 0)

    def _fill_zero(i, _):
        zbuf_v[pl.ds(i * 16, 16)] = jnp.zeros((16,), jnp.float32)
        return 0

    lax.fori_loop(0, RPS // 16, _fill_zero, 0)
    pltpu.sync_copy(zbuf_v, deg_sh.at[pl.ds(s * RPS, RPS)])
    plsc.subcore_barrier()

    def _step(g, _):
        pltpu.sync_copy(ones_v, deg_sh.at[dstb_v.at[g]], add=True)
        return 0

    lax.fori_loop(0, NCB, _step, 0)
    plsc.subcore_barrier()
    pltpu.sync_copy(deg_sh.at[pl.ds(s * RPS, RPS)],
                    out_hbm.at[c, pl.ds(s * RPS, RPS)])


def _make_scatter(dk):
    @functools.partial(
        pl.kernel,
        out_type=jax.ShapeDtypeStruct((NC, NPAD, dk), jnp.float32),
        mesh=_mesh,
        compiler_params=pltpu.CompilerParams(
            use_tc_tiling_on_sc=(dk % 128 == 0)),
        scratch_types=[
            pltpu.VMEM((IB, CHB), jnp.int32),
            pltpu.VMEM((IB, CHB), jnp.int32),
            pltpu.VMEM((CHB, dk), jnp.float32),
            pltpu.VMEM((CHB, dk), jnp.float32),
            pltpu.VMEM_SHARED((NPAD, dk), jnp.float32),
            pltpu.SemaphoreType.DMA,
            pltpu.SemaphoreType.DMA,
        ],
    )
    def _scat(y_hbm, src_hbm, dst_hbm, out_hbm,
              srcb_v, dstb_v, buf0, buf1, acc_sh, sem0, sem1):
        c = lax.axis_index("c")
        s = lax.axis_index("s")
        wid = c * NS + s

        def _zrow(i, _):
            for j in range(dk // 16):
                buf0[i, pl.ds(j * 16, 16)] = jnp.zeros((16,), jnp.float32)
            return 0

        lax.fori_loop(0, CHB, _zrow, 0)
        for k in range(RPS // CHB):
            pltpu.sync_copy(buf0, acc_sh.at[pl.ds(s * RPS + k * CHB, CHB)])
        plsc.subcore_barrier()

        def _blk(b, _):
            pltpu.sync_copy(src_hbm.at[pl.ds(wid * NCB + b * IB, IB), :],
                            srcb_v)
            pltpu.sync_copy(dst_hbm.at[pl.ds(wid * NCB + b * IB, IB), :],
                            dstb_v)
            pltpu.async_copy(y_hbm.at[srcb_v.at[0]], buf0, sem0)

            def _step(t, _):
                g0 = 2 * t
                g1 = 2 * t + 1
                pltpu.make_async_copy(
                    y_hbm.at[srcb_v.at[g0]], buf0, sem0).wait()
                pltpu.async_copy(y_hbm.at[srcb_v.at[g1]], buf1, sem1)
                pltpu.sync_copy(buf0, acc_sh.at[dstb_v.at[g0]], add=True)
                pltpu.make_async_copy(
                    y_hbm.at[srcb_v.at[g1]], buf1, sem1).wait()

                @pl.when(g1 + 1 < IB)
                def _():
                    pltpu.async_copy(y_hbm.at[srcb_v.at[g1 + 1]], buf0, sem0)

                pltpu.sync_copy(buf1, acc_sh.at[dstb_v.at[g1]], add=True)
                return 0

            lax.fori_loop(0, IB // 2, _step, 0)
            return 0

        lax.fori_loop(0, NCB // IB, _blk, 0)
        plsc.subcore_barrier()
        pltpu.sync_copy(acc_sh.at[pl.ds(s * RPS, RPS)],
                        out_hbm.at[c, pl.ds(s * RPS, RPS)])

    return _scat


_scatter128 = _make_scatter(D)
_scatter64 = _make_scatter(H)


def _ka_body(deg_ref, x_ref, w1_ref, y1_ref, dinv_ref):
    deg = deg_ref[...]
    dinv = lax.rsqrt(deg[:, 0:1] + deg[:, 1:2] + 1.0)
    xw = jnp.dot(x_ref[...], w1_ref[...], preferred_element_type=jnp.float32)
    y1_ref[...] = dinv * xw
    dinv_ref[...] = dinv


def _ka(degt, x_pad, W1):
    return pl.pallas_call(
        _ka_body,
        grid=(NB,),
        in_specs=[
            pl.BlockSpec((R, 2), lambda i: (i, 0)),
            pl.BlockSpec((R, D), lambda i: (i, 0)),
            pl.BlockSpec((D, D), lambda i: (0, 0)),
        ],
        out_specs=[
            pl.BlockSpec((R, D), lambda i: (i, 0)),
            pl.BlockSpec((R, 1), lambda i: (i, 0)),
        ],
        out_shape=[
            jax.ShapeDtypeStruct((NPAD, D), jnp.float32),
            jax.ShapeDtypeStruct((NPAD, 1), jnp.float32),
        ],
    )(degt, x_pad, W1)


def _kb_body(p_ref, y1_ref, dinv_ref, b1_ref, w2_ref, y2_ref):
    p = p_ref[...]
    dinv = dinv_ref[...]
    h1 = jnp.maximum(dinv * (p[0] + p[1] + y1_ref[...]) + b1_ref[...], 0.0)
    y2_ref[...] = dinv * jnp.dot(h1, w2_ref[...],
                                 preferred_element_type=jnp.float32)


def _kb(p, y1, dinv, b1r, W2):
    return pl.pallas_call(
        _kb_body,
        grid=(NB,),
        in_specs=[
            pl.BlockSpec((NC, R, D), lambda i: (0, i, 0)),
            pl.BlockSpec((R, D), lambda i: (i, 0)),
            pl.BlockSpec((R, 1), lambda i: (i, 0)),
            pl.BlockSpec((1, D), lambda i: (0, 0)),
            pl.BlockSpec((D, H), lambda i: (0, 0)),
        ],
        out_specs=pl.BlockSpec((R, H), lambda i: (i, 0)),
        out_shape=jax.ShapeDtypeStruct((NPAD, H), jnp.float32),
    )(p, y1, dinv, b1r, W2)


def _kc_body(q_ref, y2_ref, dinv_ref, b2_ref, batch_ref, w3_ref, b3_ref,
             out_ref, psum):
    i = pl.program_id(0)

    @pl.when(i == 0)
    def _():
        psum[...] = jnp.zeros_like(psum)

    q = q_ref[...]
    dinv = dinv_ref[...]
    h2 = jnp.maximum(dinv * (q[0] + q[1] + y2_ref[...]) + b2_ref[...], 0.0)
    bb = batch_ref[...]
    gid = lax.broadcasted_iota(jnp.int32, (1, G), 1)
    m = (bb == gid).astype(jnp.float32)
    haug = jnp.concatenate([h2, jnp.ones((R, 1), jnp.float32)], axis=1)
    psum[...] += lax.dot_general(m, haug, (((0,), (0,)), ((), ())),
                                 preferred_element_type=jnp.float32)

    @pl.when(i == pl.num_programs(0) - 1)
    def _():
        ps = psum[...]
        pooled = ps[:, :H] / jnp.maximum(ps[:, H:H + 1], 1.0)
        out_ref[...] = jnp.dot(pooled, w3_ref[...],
                               preferred_element_type=jnp.float32) + b3_ref[...]


def _kc(q, y2, dinv, b2r, batch_pad, W3, b3r):
    return pl.pallas_call(
        _kc_body,
        grid=(NB,),
        in_specs=[
            pl.BlockSpec((NC, R, H), lambda i: (0, i, 0)),
            pl.BlockSpec((R, H), lambda i: (i, 0)),
            pl.BlockSpec((R, 1), lambda i: (i, 0)),
            pl.BlockSpec((1, H), lambda i: (0, 0)),
            pl.BlockSpec((R, 1), lambda i: (i, 0)),
            pl.BlockSpec((H, C), lambda i: (0, 0)),
            pl.BlockSpec((1, C), lambda i: (0, 0)),
        ],
        out_specs=pl.BlockSpec((G, C), lambda i: (0, 0)),
        out_shape=jax.ShapeDtypeStruct((G, C), jnp.float32),
        scratch_shapes=[pltpu.VMEM((G, H + 1), jnp.float32)],
    )(q, y2, dinv, b2r, batch_pad, W3, b3r)


def kernel(x, edge_index, batch, W1, b1, W2, b2, W3, b3):
    src = edge_index[0]
    dst = edge_index[1]
    pad_ids = (jnp.arange(EPAD - E, dtype=src.dtype) % (NPAD - N)) + N
    srcp = jnp.concatenate([src, pad_ids]).reshape(NW * NCB, CHB)
    dstp = jnp.concatenate([dst, pad_ids]).reshape(NW * NCB, CHB)
    x_pad = jnp.pad(x, ((0, NPAD - N), (0, 0)))
    batch_pad = jnp.pad(batch, (0, NPAD - N), constant_values=G)
    batch_pad = batch_pad.reshape(NPAD, 1)

    degp = _deg_sc(dstp)
    degt = degp.T
    y1, dinv = _ka(degt, x_pad, W1)
    p = _scatter128(y1, srcp, dstp)
    y2 = _kb(p, y1, dinv, b1.reshape(1, D), W2)
    q = _scatter64(y2, srcp, dstp)
    return _kc(q, y2, dinv, b2.reshape(1, H), batch_pad, W3,
               b3.reshape(1, C))


# speedup vs baseline: 2.9239x; 1.0009x over previous
import functools

import jax
import jax.numpy as jnp
import numpy as np
from jax import lax
from jax.experimental import pallas as pl
from jax.experimental.pallas import tpu as pltpu
from jax.experimental.pallas import tpu_sc as plsc

N = 10000
E = 320000
D = 128
H = 64
C = 32
G = 64

NC = 2
NS = 16
NW = NC * NS
NPAD = 10240
RPS = NPAD // NS
CHB = 128
NCB = 80
IB = 16
EW = NCB * CHB
EPAD = NW * EW

R = 2048
NB = NPAD // R

_mesh = plsc.VectorSubcoreMesh(core_axis_name="c", subcore_axis_name="s")

_PAD_IDS = (np.arange(EPAD - E, dtype=np.int32) % (NPAD - N) + N)


@functools.partial(
    pl.kernel,
    out_type=jax.ShapeDtypeStruct((NC, NPAD), jnp.float32),
    mesh=_mesh,
    scratch_types=[
        pltpu.VMEM((NCB, CHB), jnp.int32),
        pltpu.VMEM((CHB,), jnp.float32),
        pltpu.VMEM((RPS,), jnp.float32),
        pltpu.VMEM_SHARED((NPAD,), jnp.float32),
    ],
)
def _deg_sc(dst_hbm, out_hbm, dstb_v, ones_v, zbuf_v, deg_sh):
    c = lax.axis_index("c")
    s = lax.axis_index("s")
    wid = c * NS + s

    pltpu.sync_copy(dst_hbm.at[pl.ds(wid * NCB, NCB), :], dstb_v)

    def _fill_ones(i, _):
        ones_v[pl.ds(i * 16, 16)] = jnp.ones((16,), jnp.float32)
        return 0

    lax.fori_loop(0, CHB // 16, _fill_ones, 0)

    def _fill_zero(i, _):
        zbuf_v[pl.ds(i * 16, 16)] = jnp.zeros((16,), jnp.float32)
        return 0

    lax.fori_loop(0, RPS // 16, _fill_zero, 0)
    pltpu.sync_copy(zbuf_v, deg_sh.at[pl.ds(s * RPS, RPS)])
    plsc.subcore_barrier()

    def _step(g, _):
        pltpu.sync_copy(ones_v, deg_sh.at[dstb_v.at[g]], add=True)
        return 0

    lax.fori_loop(0, NCB, _step, 0)
    plsc.subcore_barrier()
    pltpu.sync_copy(deg_sh.at[pl.ds(s * RPS, RPS)],
                    out_hbm.at[c, pl.ds(s * RPS, RPS)])


def _make_scatter(dk, ib):
    nblk = NCB // ib
    scratch = [
        pltpu.VMEM((ib, CHB), jnp.int32),
        pltpu.VMEM((ib, CHB), jnp.int32),
        pltpu.VMEM((CHB, dk), jnp.float32),
        pltpu.VMEM((CHB, dk), jnp.float32),
        pltpu.VMEM_SHARED((NPAD, dk), jnp.float32),
        pltpu.SemaphoreType.DMA,
        pltpu.SemaphoreType.DMA,
        pltpu.SemaphoreType.DMA,
        pltpu.SemaphoreType.DMA,
    ]
    if nblk > 1:
        scratch += [
            pltpu.VMEM((ib, CHB), jnp.int32),
            pltpu.VMEM((ib, CHB), jnp.int32),
            pltpu.SemaphoreType.DMA,
            pltpu.SemaphoreType.DMA,
        ]

    @functools.partial(
        pl.kernel,
        out_type=jax.ShapeDtypeStruct((NC, NPAD, dk), jnp.float32),
        mesh=_mesh,
        compiler_params=pltpu.CompilerParams(
            use_tc_tiling_on_sc=(dk % 128 == 0)),
        scratch_types=scratch,
    )
    def _scat(y_hbm, src_hbm, dst_hbm, out_hbm,
              srcb0, dstb0, buf0, buf1, acc_sh,
              semg0, semg1, sems0, sems1,
              srcb1=None, dstb1=None, semis=None, semid=None):
        c = lax.axis_index("c")
        s = lax.axis_index("s")
        wid = c * NS + s

        def _zrow(i, _):
            for j in range(dk // 16):
                buf0[i, pl.ds(j * 16, 16)] = jnp.zeros((16,), jnp.float32)
            return 0

        lax.fori_loop(0, CHB, _zrow, 0)
        for k in range(RPS // CHB):
            pltpu.sync_copy(buf0, acc_sh.at[pl.ds(s * RPS + k * CHB, CHB)])
        plsc.subcore_barrier()

        idxbufs = [(srcb0, dstb0), (srcb1, dstb1)]
        pltpu.sync_copy(src_hbm.at[pl.ds(wid * NCB, ib), :], srcb0)
        pltpu.sync_copy(dst_hbm.at[pl.ds(wid * NCB, ib), :], dstb0)

        for b in range(nblk):
            sb, db = idxbufs[b % 2]
            if b + 1 < nblk:
                nsb, ndb = idxbufs[(b + 1) % 2]
                pltpu.async_copy(
                    src_hbm.at[pl.ds(wid * NCB + (b + 1) * ib, ib), :],
                    nsb, semis)
                pltpu.async_copy(
                    dst_hbm.at[pl.ds(wid * NCB + (b + 1) * ib, ib), :],
                    ndb, semid)
            pltpu.async_copy(y_hbm.at[sb.at[0]], buf0, semg0)
            pltpu.async_copy(y_hbm.at[sb.at[1]], buf1, semg1)

            def _step(t, _, sb=sb, db=db):
                g0 = 2 * t
                g1 = 2 * t + 1
                pltpu.make_async_copy(
                    y_hbm.at[sb.at[g0]], buf0, semg0).wait()
                pltpu.async_copy(buf0, acc_sh.at[db.at[g0]], sems0, add=True)
                pltpu.make_async_copy(
                    y_hbm.at[sb.at[g1]], buf1, semg1).wait()
                pltpu.async_copy(buf1, acc_sh.at[db.at[g1]], sems1, add=True)

                @pl.when(t + 1 < ib // 2)
                def _():
                    pltpu.make_async_copy(
                        buf0, acc_sh.at[db.at[g0]], sems0).wait()
                    pltpu.async_copy(y_hbm.at[sb.at[g0 + 2]], buf0, semg0)
                    pltpu.make_async_copy(
                        buf1, acc_sh.at[db.at[g1]], sems1).wait()
                    pltpu.async_copy(y_hbm.at[sb.at[g1 + 2]], buf1, semg1)

                return 0

            lax.fori_loop(0, ib // 2, _step, 0)
            pltpu.make_async_copy(buf0, acc_sh.at[db.at[ib - 2]],
                                  sems0).wait()
            pltpu.make_async_copy(buf1, acc_sh.at[db.at[ib - 1]],
                                  sems1).wait()
            if b + 1 < nblk:
                pltpu.make_async_copy(
                    src_hbm.at[pl.ds(wid * NCB + (b + 1) * ib, ib), :],
                    nsb, semis).wait()
                pltpu.make_async_copy(
                    dst_hbm.at[pl.ds(wid * NCB + (b + 1) * ib, ib), :],
                    ndb, semid).wait()

        plsc.subcore_barrier()
        pltpu.sync_copy(acc_sh.at[pl.ds(s * RPS, RPS)],
                        out_hbm.at[c, pl.ds(s * RPS, RPS)])

    return _scat


_scatter128 = _make_scatter(D, 16)
_scatter64 = _make_scatter(H, 80)


def _ka_body(deg_ref, x_ref, w1_ref, y1_ref, dinv_ref):
    deg = deg_ref[...]
    dinv = lax.rsqrt(deg[:, 0:1] + deg[:, 1:2] + 1.0)
    xw = jnp.dot(x_ref[...], w1_ref[...], preferred_element_type=jnp.float32)
    y1_ref[...] = dinv * xw
    dinv_ref[...] = dinv


def _ka(degt, x_pad, W1):
    return pl.pallas_call(
        _ka_body,
        grid=(NB,),
        in_specs=[
            pl.BlockSpec((R, 2), lambda i: (i, 0)),
            pl.BlockSpec((R, D), lambda i: (i, 0)),
            pl.BlockSpec((D, D), lambda i: (0, 0)),
        ],
        out_specs=[
            pl.BlockSpec((R, D), lambda i: (i, 0)),
            pl.BlockSpec((R, 1), lambda i: (i, 0)),
        ],
        out_shape=[
            jax.ShapeDtypeStruct((NPAD, D), jnp.float32),
            jax.ShapeDtypeStruct((NPAD, 1), jnp.float32),
        ],
    )(degt, x_pad, W1)


def _kb_body(p_ref, y1_ref, dinv_ref, b1_ref, w2_ref, y2_ref):
    p = p_ref[...]
    dinv = dinv_ref[...]
    h1 = jnp.maximum(dinv * (p[0] + p[1] + y1_ref[...]) + b1_ref[...], 0.0)
    y2_ref[...] = dinv * jnp.dot(h1, w2_ref[...],
                                 preferred_element_type=jnp.float32)


def _kb(p, y1, dinv, b1r, W2):
    return pl.pallas_call(
        _kb_body,
        grid=(NB,),
        in_specs=[
            pl.BlockSpec((NC, R, D), lambda i: (0, i, 0)),
            pl.BlockSpec((R, D), lambda i: (i, 0)),
            pl.BlockSpec((R, 1), lambda i: (i, 0)),
            pl.BlockSpec((1, D), lambda i: (0, 0)),
            pl.BlockSpec((D, H), lambda i: (0, 0)),
        ],
        out_specs=pl.BlockSpec((R, H), lambda i: (i, 0)),
        out_shape=jax.ShapeDtypeStruct((NPAD, H), jnp.float32),
    )(p, y1, dinv, b1r, W2)


def _kc_body(q_ref, y2_ref, dinv_ref, b2_ref, batch_ref, w3_ref, b3_ref,
             out_ref, psum):
    i = pl.program_id(0)

    @pl.when(i == 0)
    def _():
        psum[...] = jnp.zeros_like(psum)

    q = q_ref[...]
    dinv = dinv_ref[...]
    h2 = jnp.maximum(dinv * (q[0] + q[1] + y2_ref[...]) + b2_ref[...], 0.0)
    bb = batch_ref[...]
    gid = lax.broadcasted_iota(jnp.int32, (1, G), 1)
    m = (bb == gid).astype(jnp.float32)
    haug = jnp.concatenate([h2, jnp.ones((R, 1), jnp.float32)], axis=1)
    psum[...] += lax.dot_general(m, haug, (((0,), (0,)), ((), ())),
                                 preferred_element_type=jnp.float32)

    @pl.when(i == pl.num_programs(0) - 1)
    def _():
        ps = psum[...]
        pooled = ps[:, :H] / jnp.maximum(ps[:, H:H + 1], 1.0)
        out_ref[...] = jnp.dot(pooled, w3_ref[...],
                               preferred_element_type=jnp.float32) + b3_ref[...]


def _kc(q, y2, dinv, b2r, batch_pad, W3, b3r):
    return pl.pallas_call(
        _kc_body,
        grid=(NB,),
        in_specs=[
            pl.BlockSpec((NC, R, H), lambda i: (0, i, 0)),
            pl.BlockSpec((R, H), lambda i: (i, 0)),
            pl.BlockSpec((R, 1), lambda i: (i, 0)),
            pl.BlockSpec((1, H), lambda i: (0, 0)),
            pl.BlockSpec((R, 1), lambda i: (i, 0)),
            pl.BlockSpec((H, C), lambda i: (0, 0)),
            pl.BlockSpec((1, C), lambda i: (0, 0)),
        ],
        out_specs=pl.BlockSpec((G, C), lambda i: (0, 0)),
        out_shape=jax.ShapeDtypeStruct((G, C), jnp.float32),
        scratch_shapes=[pltpu.VMEM((G, H + 1), jnp.float32)],
    )(q, y2, dinv, b2r, batch_pad, W3, b3r)


def kernel(x, edge_index, batch, W1, b1, W2, b2, W3, b3):
    src = edge_index[0]
    dst = edge_index[1]
    pad_ids = jnp.asarray(_PAD_IDS)
    srcp = jnp.concatenate([src, pad_ids]).reshape(NW * NCB, CHB)
    dstp = jnp.concatenate([dst, pad_ids]).reshape(NW * NCB, CHB)
    x_pad = jnp.pad(x, ((0, NPAD - N), (0, 0)))
    batch_pad = jnp.pad(batch, (0, NPAD - N), constant_values=G)
    batch_pad = batch_pad.reshape(NPAD, 1)

    degp = _deg_sc(dstp)
    degt = degp.T
    y1, dinv = _ka(degt, x_pad, W1)
    p = _scatter128(y1, srcp, dstp)
    y2 = _kb(p, y1, dinv, b1.reshape(1, D), W2)
    q = _scatter64(y2, srcp, dstp)
    return _kc(q, y2, dinv, b2.reshape(1, H), batch_pad, W3,
               b3.reshape(1, C))


# speedup vs baseline: 3.0490x; 1.0428x over previous
import functools

import jax
import jax.numpy as jnp
import numpy as np
from jax import lax
from jax.experimental import pallas as pl
from jax.experimental.pallas import tpu as pltpu
from jax.experimental.pallas import tpu_sc as plsc

N = 10000
E = 320000
D = 128
H = 64
C = 32
G = 64

NC = 2
NS = 16
NW = NC * NS
NPAD = 10240
RPS = NPAD // NS
CHB = 128
NCB = 80
IB = 16
EW = NCB * CHB
EPAD = NW * EW

R = 2048
NB = NPAD // R

_mesh = plsc.VectorSubcoreMesh(core_axis_name="c", subcore_axis_name="s")

_PAD_IDS = (np.arange(EPAD - E, dtype=np.int32) % (NPAD - N) + N)


@functools.partial(
    pl.kernel,
    out_type=jax.ShapeDtypeStruct((NC, NPAD), jnp.float32),
    mesh=_mesh,
    scratch_types=[
        pltpu.VMEM((NCB, CHB), jnp.int32),
        pltpu.VMEM((CHB,), jnp.float32),
        pltpu.VMEM((RPS,), jnp.float32),
        pltpu.VMEM_SHARED((NPAD,), jnp.float32),
    ],
)
def _deg_sc(dst_hbm, out_hbm, dstb_v, ones_v, zbuf_v, deg_sh):
    c = lax.axis_index("c")
    s = lax.axis_index("s")
    wid = c * NS + s

    pltpu.sync_copy(dst_hbm.at[pl.ds(wid * NCB, NCB), :], dstb_v)

    def _fill_ones(i, _):
        ones_v[pl.ds(i * 16, 16)] = jnp.ones((16,), jnp.float32)
        return 0

    lax.fori_loop(0, CHB // 16, _fill_ones, 0)

    def _fill_zero(i, _):
        zbuf_v[pl.ds(i * 16, 16)] = jnp.zeros((16,), jnp.float32)
        return 0

    lax.fori_loop(0, RPS // 16, _fill_zero, 0)
    pltpu.sync_copy(zbuf_v, deg_sh.at[pl.ds(s * RPS, RPS)])
    plsc.subcore_barrier()

    def _step(g, _):
        pltpu.sync_copy(ones_v, deg_sh.at[dstb_v.at[g]], add=True)
        return 0

    lax.fori_loop(0, NCB, _step, 0)
    plsc.subcore_barrier()
    pltpu.sync_copy(deg_sh.at[pl.ds(s * RPS, RPS)],
                    out_hbm.at[c, pl.ds(s * RPS, RPS)])


def _make_scatter(dk, ib):
    nblk = NCB // ib
    scratch = [
        pltpu.VMEM((ib, CHB), jnp.int32),
        pltpu.VMEM((ib, CHB), jnp.int32),
        pltpu.VMEM((CHB, dk), jnp.float32),
        pltpu.VMEM((CHB, dk), jnp.float32),
        pltpu.VMEM_SHARED((NPAD, dk), jnp.float32),
        pltpu.SemaphoreType.DMA,
        pltpu.SemaphoreType.DMA,
        pltpu.SemaphoreType.DMA,
        pltpu.SemaphoreType.DMA,
    ]
    if nblk > 1:
        scratch += [
            pltpu.VMEM((ib, CHB), jnp.int32),
            pltpu.VMEM((ib, CHB), jnp.int32),
            pltpu.SemaphoreType.DMA,
            pltpu.SemaphoreType.DMA,
        ]

    @functools.partial(
        pl.kernel,
        out_type=jax.ShapeDtypeStruct((NC, NPAD, dk), jnp.float32),
        mesh=_mesh,
        compiler_params=pltpu.CompilerParams(
            use_tc_tiling_on_sc=(dk % 128 == 0)),
        scratch_types=scratch,
    )
    def _scat(y_hbm, src_hbm, dst_hbm, out_hbm,
              srcb0, dstb0, buf0, buf1, acc_sh,
              semg0, semg1, sems0, sems1,
              srcb1=None, dstb1=None, semis=None, semid=None):
        c = lax.axis_index("c")
        s = lax.axis_index("s")
        wid = c * NS + s

        def _zrow(i, _):
            for j in range(dk // 16):
                buf0[i, pl.ds(j * 16, 16)] = jnp.zeros((16,), jnp.float32)
            return 0

        lax.fori_loop(0, CHB, _zrow, 0)
        for k in range(RPS // CHB):
            pltpu.sync_copy(buf0, acc_sh.at[pl.ds(s * RPS + k * CHB, CHB)])
        plsc.subcore_barrier()

        idxbufs = [(srcb0, dstb0), (srcb1, dstb1)]
        pltpu.sync_copy(src_hbm.at[pl.ds(wid * NCB, ib), :], srcb0)
        pltpu.sync_copy(dst_hbm.at[pl.ds(wid * NCB, ib), :], dstb0)

        for b in range(nblk):
            sb, db = idxbufs[b % 2]
            if b + 1 < nblk:
                nsb, ndb = idxbufs[(b + 1) % 2]
                pltpu.async_copy(
                    src_hbm.at[pl.ds(wid * NCB + (b + 1) * ib, ib), :],
                    nsb, semis)
                pltpu.async_copy(
                    dst_hbm.at[pl.ds(wid * NCB + (b + 1) * ib, ib), :],
                    ndb, semid)
            pltpu.async_copy(y_hbm.at[sb.at[0]], buf0, semg0)

            if nblk > 1:
                def _step(t, _, sb=sb, db=db):
                    g0 = 2 * t
                    g1 = 2 * t + 1
                    pltpu.make_async_copy(
                        y_hbm.at[sb.at[g0]], buf0, semg0).wait()
                    pltpu.async_copy(y_hbm.at[sb.at[g1]], buf1, semg1)
                    pltpu.sync_copy(buf0, acc_sh.at[db.at[g0]], add=True)
                    pltpu.make_async_copy(
                        y_hbm.at[sb.at[g1]], buf1, semg1).wait()

                    @pl.when(g1 + 1 < ib)
                    def _():
                        pltpu.async_copy(y_hbm.at[sb.at[g1 + 1]], buf0, semg0)

                    pltpu.sync_copy(buf1, acc_sh.at[db.at[g1]], add=True)
                    return 0

                lax.fori_loop(0, ib // 2, _step, 0)
            else:
                pltpu.async_copy(y_hbm.at[sb.at[1]], buf1, semg1)

                def _step(t, _, sb=sb, db=db):
                    g0 = 2 * t
                    g1 = 2 * t + 1
                    pltpu.make_async_copy(
                        y_hbm.at[sb.at[g0]], buf0, semg0).wait()
                    pltpu.async_copy(buf0, acc_sh.at[db.at[g0]], sems0,
                                     add=True)
                    pltpu.make_async_copy(
                        y_hbm.at[sb.at[g1]], buf1, semg1).wait()
                    pltpu.async_copy(buf1, acc_sh.at[db.at[g1]], sems1,
                                     add=True)

                    @pl.when(t + 1 < ib // 2)
                    def _():
                        pltpu.make_async_copy(
                            buf0, acc_sh.at[db.at[g0]], sems0).wait()
                        pltpu.async_copy(y_hbm.at[sb.at[g0 + 2]], buf0, semg0)
                        pltpu.make_async_copy(
                            buf1, acc_sh.at[db.at[g1]], sems1).wait()
                        pltpu.async_copy(y_hbm.at[sb.at[g1 + 2]], buf1, semg1)

                    return 0

                lax.fori_loop(0, ib // 2, _step, 0)
                pltpu.make_async_copy(buf0, acc_sh.at[db.at[ib - 2]],
                                      sems0).wait()
                pltpu.make_async_copy(buf1, acc_sh.at[db.at[ib - 1]],
                                      sems1).wait()
            if b + 1 < nblk:
                pltpu.make_async_copy(
                    src_hbm.at[pl.ds(wid * NCB + (b + 1) * ib, ib), :],
                    nsb, semis).wait()
                pltpu.make_async_copy(
                    dst_hbm.at[pl.ds(wid * NCB + (b + 1) * ib, ib), :],
                    ndb, semid).wait()

        plsc.subcore_barrier()
        pltpu.sync_copy(acc_sh.at[pl.ds(s * RPS, RPS)],
                        out_hbm.at[c, pl.ds(s * RPS, RPS)])

    return _scat


_scatter128 = _make_scatter(D, 16)
_scatter64 = _make_scatter(H, 80)


def _ka_body(deg_ref, x_ref, w1_ref, y1_ref, dinv_ref):
    deg = deg_ref[...]
    dinv = lax.rsqrt(deg[:, 0:1] + deg[:, 1:2] + 1.0)
    xw = jnp.dot(x_ref[...], w1_ref[...], preferred_element_type=jnp.float32)
    y1_ref[...] = dinv * xw
    dinv_ref[...] = dinv


def _ka(degt, x_pad, W1):
    return pl.pallas_call(
        _ka_body,
        grid=(NB,),
        in_specs=[
            pl.BlockSpec((R, 2), lambda i: (i, 0)),
            pl.BlockSpec((R, D), lambda i: (i, 0)),
            pl.BlockSpec((D, D), lambda i: (0, 0)),
        ],
        out_specs=[
            pl.BlockSpec((R, D), lambda i: (i, 0)),
            pl.BlockSpec((R, 1), lambda i: (i, 0)),
        ],
        out_shape=[
            jax.ShapeDtypeStruct((NPAD, D), jnp.float32),
            jax.ShapeDtypeStruct((NPAD, 1), jnp.float32),
        ],
    )(degt, x_pad, W1)


def _kb_body(p_ref, y1_ref, dinv_ref, b1_ref, w2_ref, y2_ref):
    p = p_ref[...]
    dinv = dinv_ref[...]
    h1 = jnp.maximum(dinv * (p[0] + p[1] + y1_ref[...]) + b1_ref[...], 0.0)
    y2_ref[...] = dinv * jnp.dot(h1, w2_ref[...],
                                 preferred_element_type=jnp.float32)


def _kb(p, y1, dinv, b1r, W2):
    return pl.pallas_call(
        _kb_body,
        grid=(NB,),
        in_specs=[
            pl.BlockSpec((NC, R, D), lambda i: (0, i, 0)),
            pl.BlockSpec((R, D), lambda i: (i, 0)),
            pl.BlockSpec((R, 1), lambda i: (i, 0)),
            pl.BlockSpec((1, D), lambda i: (0, 0)),
            pl.BlockSpec((D, H), lambda i: (0, 0)),
        ],
        out_specs=pl.BlockSpec((R, H), lambda i: (i, 0)),
        out_shape=jax.ShapeDtypeStruct((NPAD, H), jnp.float32),
    )(p, y1, dinv, b1r, W2)


def _kc_body(q_ref, y2_ref, dinv_ref, b2_ref, batch_ref, w3_ref, b3_ref,
             out_ref, psum):
    i = pl.program_id(0)

    @pl.when(i == 0)
    def _():
        psum[...] = jnp.zeros_like(psum)

    q = q_ref[...]
    dinv = dinv_ref[...]
    h2 = jnp.maximum(dinv * (q[0] + q[1] + y2_ref[...]) + b2_ref[...], 0.0)
    bb = batch_ref[...]
    gid = lax.broadcasted_iota(jnp.int32, (1, G), 1)
    m = (bb == gid).astype(jnp.float32)
    haug = jnp.concatenate([h2, jnp.ones((R, 1), jnp.float32)], axis=1)
    psum[...] += lax.dot_general(m, haug, (((0,), (0,)), ((), ())),
                                 preferred_element_type=jnp.float32)

    @pl.when(i == pl.num_programs(0) - 1)
    def _():
        ps = psum[...]
        pooled = ps[:, :H] / jnp.maximum(ps[:, H:H + 1], 1.0)
        out_ref[...] = jnp.dot(pooled, w3_ref[...],
                               preferred_element_type=jnp.float32) + b3_ref[...]


def _kc(q, y2, dinv, b2r, batch_pad, W3, b3r):
    return pl.pallas_call(
        _kc_body,
        grid=(NB,),
        in_specs=[
            pl.BlockSpec((NC, R, H), lambda i: (0, i, 0)),
            pl.BlockSpec((R, H), lambda i: (i, 0)),
            pl.BlockSpec((R, 1), lambda i: (i, 0)),
            pl.BlockSpec((1, H), lambda i: (0, 0)),
            pl.BlockSpec((R, 1), lambda i: (i, 0)),
            pl.BlockSpec((H, C), lambda i: (0, 0)),
            pl.BlockSpec((1, C), lambda i: (0, 0)),
        ],
        out_specs=pl.BlockSpec((G, C), lambda i: (0, 0)),
        out_shape=jax.ShapeDtypeStruct((G, C), jnp.float32),
        scratch_shapes=[pltpu.VMEM((G, H + 1), jnp.float32)],
    )(q, y2, dinv, b2r, batch_pad, W3, b3r)


def kernel(x, edge_index, batch, W1, b1, W2, b2, W3, b3):
    src = edge_index[0]
    dst = edge_index[1]
    pad_ids = jnp.asarray(_PAD_IDS)
    srcp = jnp.concatenate([src, pad_ids]).reshape(NW * NCB, CHB)
    dstp = jnp.concatenate([dst, pad_ids]).reshape(NW * NCB, CHB)
    x_pad = jnp.pad(x, ((0, NPAD - N), (0, 0)))
    batch_pad = jnp.pad(batch, (0, NPAD - N), constant_values=G)
    batch_pad = batch_pad.reshape(NPAD, 1)

    degp = _deg_sc(dstp)
    y1, dinv = _ka(degp.T, x_pad, W1)
    p = _scatter128(y1, srcp, dstp)
    y2 = _kb(p, y1, dinv, b1.reshape(1, D), W2)
    q = _scatter64(y2, srcp, dstp)
    return _kc(q, y2, dinv, b2.reshape(1, H), batch_pad, W3,
               b3.reshape(1, C))


# speedup vs baseline: 3.1236x; 1.0245x over previous
import functools

import jax
import jax.numpy as jnp
import numpy as np
from jax import lax
from jax.experimental import pallas as pl
from jax.experimental.pallas import tpu as pltpu
from jax.experimental.pallas import tpu_sc as plsc

N = 10000
E = 320000
D = 128
H = 64
C = 32
G = 64

NC = 2
NS = 16
NW = NC * NS
NPAD = 10240
RPS = NPAD // NS
CHB = 128
NCB = 80
IB = 16
EW = NCB * CHB
EPAD = NW * EW

R = 2048
NB = NPAD // R

_mesh = plsc.VectorSubcoreMesh(core_axis_name="c", subcore_axis_name="s")

_PAD_IDS = (np.arange(EPAD - E, dtype=np.int32) % (NPAD - N) + N)


@functools.partial(
    pl.kernel,
    out_type=jax.ShapeDtypeStruct((NC, NPAD), jnp.float32),
    mesh=_mesh,
    scratch_types=[
        pltpu.VMEM((NCB, CHB), jnp.int32),
        pltpu.VMEM((CHB,), jnp.float32),
        pltpu.VMEM((RPS,), jnp.float32),
        pltpu.VMEM_SHARED((NPAD,), jnp.float32),
    ],
)
def _deg_sc(edges_hbm, out_hbm, dstb_v, ones_v, zbuf_v, deg_sh):
    c = lax.axis_index("c")
    s = lax.axis_index("s")
    wid = c * NS + s

    pltpu.sync_copy(edges_hbm.at[1, pl.ds(wid * NCB, NCB), :], dstb_v)

    def _fill_ones(i, _):
        ones_v[pl.ds(i * 16, 16)] = jnp.ones((16,), jnp.float32)
        return 0

    lax.fori_loop(0, CHB // 16, _fill_ones, 0)

    def _fill_zero(i, _):
        zbuf_v[pl.ds(i * 16, 16)] = jnp.zeros((16,), jnp.float32)
        return 0

    lax.fori_loop(0, RPS // 16, _fill_zero, 0)
    pltpu.sync_copy(zbuf_v, deg_sh.at[pl.ds(s * RPS, RPS)])
    plsc.subcore_barrier()

    def _step(g, _):
        pltpu.sync_copy(ones_v, deg_sh.at[dstb_v.at[g]], add=True)
        return 0

    lax.fori_loop(0, NCB, _step, 0)
    plsc.subcore_barrier()
    pltpu.sync_copy(deg_sh.at[pl.ds(s * RPS, RPS)],
                    out_hbm.at[c, pl.ds(s * RPS, RPS)])


def _make_scatter(dk, ib):
    nblk = NCB // ib
    scratch = [
        pltpu.VMEM((ib, CHB), jnp.int32),
        pltpu.VMEM((ib, CHB), jnp.int32),
        pltpu.VMEM((CHB, dk), jnp.float32),
        pltpu.VMEM((CHB, dk), jnp.float32),
        pltpu.VMEM_SHARED((NPAD, dk), jnp.float32),
        pltpu.SemaphoreType.DMA,
        pltpu.SemaphoreType.DMA,
        pltpu.SemaphoreType.DMA,
        pltpu.SemaphoreType.DMA,
    ]
    if nblk > 1:
        scratch += [
            pltpu.VMEM((ib, CHB), jnp.int32),
            pltpu.VMEM((ib, CHB), jnp.int32),
            pltpu.SemaphoreType.DMA,
            pltpu.SemaphoreType.DMA,
        ]

    @functools.partial(
        pl.kernel,
        out_type=jax.ShapeDtypeStruct((NC, NPAD, dk), jnp.float32),
        mesh=_mesh,
        compiler_params=pltpu.CompilerParams(
            use_tc_tiling_on_sc=(dk % 128 == 0)),
        scratch_types=scratch,
    )
    def _scat(y_hbm, edges_hbm, out_hbm,
              srcb0, dstb0, buf0, buf1, acc_sh,
              semg0, semg1, sems0, sems1,
              srcb1=None, dstb1=None, semis=None, semid=None):
        c = lax.axis_index("c")
        s = lax.axis_index("s")
        wid = c * NS + s

        def _zrow(i, _):
            for j in range(dk // 16):
                buf0[i, pl.ds(j * 16, 16)] = jnp.zeros((16,), jnp.float32)
            return 0

        lax.fori_loop(0, CHB, _zrow, 0)
        for k in range(RPS // CHB):
            pltpu.sync_copy(buf0, acc_sh.at[pl.ds(s * RPS + k * CHB, CHB)])
        plsc.subcore_barrier()

        idxbufs = [(srcb0, dstb0), (srcb1, dstb1)]
        pltpu.sync_copy(edges_hbm.at[0, pl.ds(wid * NCB, ib), :], srcb0)
        pltpu.sync_copy(edges_hbm.at[1, pl.ds(wid * NCB, ib), :], dstb0)

        for b in range(nblk):
            sb, db = idxbufs[b % 2]
            if b + 1 < nblk:
                nsb, ndb = idxbufs[(b + 1) % 2]
                pltpu.async_copy(
                    edges_hbm.at[0, pl.ds(wid * NCB + (b + 1) * ib, ib), :],
                    nsb, semis)
                pltpu.async_copy(
                    edges_hbm.at[1, pl.ds(wid * NCB + (b + 1) * ib, ib), :],
                    ndb, semid)
            pltpu.async_copy(y_hbm.at[sb.at[0]], buf0, semg0)

            if nblk > 1:
                def _step(t, _, sb=sb, db=db):
                    g0 = 2 * t
                    g1 = 2 * t + 1
                    pltpu.make_async_copy(
                        y_hbm.at[sb.at[g0]], buf0, semg0).wait()
                    pltpu.async_copy(y_hbm.at[sb.at[g1]], buf1, semg1)
                    pltpu.sync_copy(buf0, acc_sh.at[db.at[g0]], add=True)
                    pltpu.make_async_copy(
                        y_hbm.at[sb.at[g1]], buf1, semg1).wait()

                    @pl.when(g1 + 1 < ib)
                    def _():
                        pltpu.async_copy(y_hbm.at[sb.at[g1 + 1]], buf0, semg0)

                    pltpu.sync_copy(buf1, acc_sh.at[db.at[g1]], add=True)
                    return 0

                lax.fori_loop(0, ib // 2, _step, 0)
            else:
                pltpu.async_copy(y_hbm.at[sb.at[1]], buf1, semg1)

                def _step(t, _, sb=sb, db=db):
                    g0 = 2 * t
                    g1 = 2 * t + 1
                    pltpu.make_async_copy(
                        y_hbm.at[sb.at[g0]], buf0, semg0).wait()
                    pltpu.async_copy(buf0, acc_sh.at[db.at[g0]], sems0,
                                     add=True)
                    pltpu.make_async_copy(
                        y_hbm.at[sb.at[g1]], buf1, semg1).wait()
                    pltpu.async_copy(buf1, acc_sh.at[db.at[g1]], sems1,
                                     add=True)

                    @pl.when(t + 1 < ib // 2)
                    def _():
                        pltpu.make_async_copy(
                            buf0, acc_sh.at[db.at[g0]], sems0).wait()
                        pltpu.async_copy(y_hbm.at[sb.at[g0 + 2]], buf0, semg0)
                        pltpu.make_async_copy(
                            buf1, acc_sh.at[db.at[g1]], sems1).wait()
                        pltpu.async_copy(y_hbm.at[sb.at[g1 + 2]], buf1, semg1)

                    return 0

                lax.fori_loop(0, ib // 2, _step, 0)
                pltpu.make_async_copy(buf0, acc_sh.at[db.at[ib - 2]],
                                      sems0).wait()
                pltpu.make_async_copy(buf1, acc_sh.at[db.at[ib - 1]],
                                      sems1).wait()
            if b + 1 < nblk:
                pltpu.make_async_copy(
                    edges_hbm.at[0, pl.ds(wid * NCB + (b + 1) * ib, ib), :],
                    nsb, semis).wait()
                pltpu.make_async_copy(
                    edges_hbm.at[1, pl.ds(wid * NCB + (b + 1) * ib, ib), :],
                    ndb, semid).wait()

        plsc.subcore_barrier()
        pltpu.sync_copy(acc_sh.at[pl.ds(s * RPS, RPS)],
                        out_hbm.at[c, pl.ds(s * RPS, RPS)])

    return _scat


_scatter128 = _make_scatter(D, 16)
_scatter64 = _make_scatter(H, 80)


def _ka_body(deg_ref, x_ref, w1_ref, y1_ref, dinv_ref):
    deg = deg_ref[...]
    dinv = lax.rsqrt(deg[:, 0:1] + deg[:, 1:2] + 1.0)
    xw = jnp.dot(x_ref[...], w1_ref[...], preferred_element_type=jnp.float32)
    y1_ref[...] = dinv * xw
    dinv_ref[...] = dinv


def _ka(degt, x_pad, W1):
    return pl.pallas_call(
        _ka_body,
        grid=(NB,),
        in_specs=[
            pl.BlockSpec((R, 2), lambda i: (i, 0)),
            pl.BlockSpec((R, D), lambda i: (i, 0)),
            pl.BlockSpec((D, D), lambda i: (0, 0)),
        ],
        out_specs=[
            pl.BlockSpec((R, D), lambda i: (i, 0)),
            pl.BlockSpec((R, 1), lambda i: (i, 0)),
        ],
        out_shape=[
            jax.ShapeDtypeStruct((NPAD, D), jnp.float32),
            jax.ShapeDtypeStruct((NPAD, 1), jnp.float32),
        ],
    )(degt, x_pad, W1)


def _kb_body(p_ref, y1_ref, dinv_ref, b1_ref, w2_ref, y2_ref):
    p = p_ref[...]
    dinv = dinv_ref[...]
    h1 = jnp.maximum(dinv * (p[0] + p[1] + y1_ref[...]) + b1_ref[...], 0.0)
    y2_ref[...] = dinv * jnp.dot(h1, w2_ref[...],
                                 preferred_element_type=jnp.float32)


def _kb(p, y1, dinv, b1r, W2):
    return pl.pallas_call(
        _kb_body,
        grid=(NB,),
        in_specs=[
            pl.BlockSpec((NC, R, D), lambda i: (0, i, 0)),
            pl.BlockSpec((R, D), lambda i: (i, 0)),
            pl.BlockSpec((R, 1), lambda i: (i, 0)),
            pl.BlockSpec((1, D), lambda i: (0, 0)),
            pl.BlockSpec((D, H), lambda i: (0, 0)),
        ],
        out_specs=pl.BlockSpec((R, H), lambda i: (i, 0)),
        out_shape=jax.ShapeDtypeStruct((NPAD, H), jnp.float32),
    )(p, y1, dinv, b1r, W2)


def _kc_body(q_ref, y2_ref, dinv_ref, b2_ref, batch_ref, w3_ref, b3_ref,
             out_ref, psum):
    i = pl.program_id(0)

    @pl.when(i == 0)
    def _():
        psum[...] = jnp.zeros_like(psum)

    q = q_ref[...]
    dinv = dinv_ref[...]
    h2 = jnp.maximum(dinv * (q[0] + q[1] + y2_ref[...]) + b2_ref[...], 0.0)
    bb = batch_ref[...]
    gid = lax.broadcasted_iota(jnp.int32, (1, G), 1)
    m = (bb == gid).astype(jnp.float32)
    haug = jnp.concatenate([h2, jnp.ones((R, 1), jnp.float32)], axis=1)
    psum[...] += lax.dot_general(m, haug, (((0,), (0,)), ((), ())),
                                 preferred_element_type=jnp.float32)

    @pl.when(i == pl.num_programs(0) - 1)
    def _():
        ps = psum[...]
        pooled = ps[:, :H] / jnp.maximum(ps[:, H:H + 1], 1.0)
        out_ref[...] = jnp.dot(pooled, w3_ref[...],
                               preferred_element_type=jnp.float32) + b3_ref[...]


def _kc(q, y2, dinv, b2r, batch_pad, W3, b3r):
    return pl.pallas_call(
        _kc_body,
        grid=(NB,),
        in_specs=[
            pl.BlockSpec((NC, R, H), lambda i: (0, i, 0)),
            pl.BlockSpec((R, H), lambda i: (i, 0)),
            pl.BlockSpec((R, 1), lambda i: (i, 0)),
            pl.BlockSpec((1, H), lambda i: (0, 0)),
            pl.BlockSpec((R, 1), lambda i: (i, 0)),
            pl.BlockSpec((H, C), lambda i: (0, 0)),
            pl.BlockSpec((1, C), lambda i: (0, 0)),
        ],
        out_specs=pl.BlockSpec((G, C), lambda i: (0, 0)),
        out_shape=jax.ShapeDtypeStruct((G, C), jnp.float32),
        scratch_shapes=[pltpu.VMEM((G, H + 1), jnp.float32)],
    )(q, y2, dinv, b2r, batch_pad, W3, b3r)


def kernel(x, edge_index, batch, W1, b1, W2, b2, W3, b3):
    src = edge_index[0]
    dst = edge_index[1]
    pad_ids = jnp.asarray(_PAD_IDS)[None, :]
    edges = jnp.concatenate(
        [edge_index, jnp.broadcast_to(pad_ids, (2, EPAD - E))],
        axis=1).reshape(2, NW * NCB, CHB)
    x_pad = jnp.pad(x, ((0, NPAD - N), (0, 0)))
    batch_pad = jnp.pad(batch, (0, NPAD - N), constant_values=G)
    batch_pad = batch_pad.reshape(NPAD, 1)

    degp = _deg_sc(edges)
    y1, dinv = _ka(degp.T, x_pad, W1)
    p = _scatter128(y1, edges)
    y2 = _kb(p, y1, dinv, b1.reshape(1, D), W2)
    q = _scatter64(y2, edges)
    return _kc(q, y2, dinv, b2.reshape(1, H), batch_pad, W3,
               b3.reshape(1, C))
